# baseline (speedup 1.0000x reference)
import functools

import jax
import jax.numpy as jnp
from jax import lax
from jax.experimental import pallas as pl
from jax.experimental.pallas import tpu as pltpu
from jax.experimental.pallas import tpu_sc as plsc

N_NODES = 10000
FEAT = 128
OE = 256
EDGE_BLOCK = 3200
NODE_BLOCK = 2000
CHUNK = 128
N_TILES = 32
N_PAD = 10240
SUB_ROWS = N_PAD // 16
P_PHASES = 2


def _edge_body(rbft_ref, x_ref, w_ref, h_ref):
    s = lax.dot_general(rbft_ref[...], w_ref[...],
                        dimension_numbers=(((0,), (0,)), ((), ())),
                        preferred_element_type=jnp.float32)
    h_ref[...] = s * x_ref[...]


def _mlp_body(*refs):
    p_refs = refs[:P_PHASES]
    wup_ref, w1_ref, b1_ref, w2_ref, b2_ref, w3_ref, b3_ref, wout_ref = \
        refs[P_PHASES:-1]
    o_ref = refs[-1]
    h = p_refs[0][0] + p_refs[0][1]
    for p_ref in p_refs[1:]:
        h = h + p_ref[0] + p_ref[1]
    h = jnp.dot(h, wup_ref[...], preferred_element_type=jnp.float32)
    for w_r, b_r in ((w1_ref, b1_ref), (w2_ref, b2_ref), (w3_ref, b3_ref)):
        z = jnp.dot(h, w_r[...], preferred_element_type=jnp.float32) + b_r[...]
        h = z * jax.nn.sigmoid(z)
    o_ref[...] = jnp.dot(h, wout_ref[...], preferred_element_type=jnp.float32)


def _sc_segment_sum(h, idx2d, zeros, tile_chunks):
    e = h.shape[0]
    n_chunks = e // CHUNK
    mesh = plsc.VectorSubcoreMesh(core_axis_name="c", subcore_axis_name="s")

    @functools.partial(
        pl.kernel,
        mesh=mesh,
        compiler_params=pltpu.CompilerParams(use_tc_tiling_on_sc=True),
        out_type=jax.ShapeDtypeStruct((2 * N_PAD, FEAT), jnp.float32),
        scratch_types=[
            pltpu.VMEM((2, CHUNK, FEAT), jnp.float32),
            pltpu.VMEM((tile_chunks, 128), jnp.int32),
            pltpu.VMEM_SHARED((N_PAD, FEAT), jnp.float32),
            pltpu.SemaphoreType.DMA,
            pltpu.SemaphoreType.DMA,
            pltpu.SemaphoreType.DMA,
            pltpu.SemaphoreType.DMA,
            pltpu.SemaphoreType.DMA,
        ],
    )
    def run(h_hbm, idx_hbm, z_hbm, out_hbm, hbuf, idxbuf, acc,
            sem0, sem1, ssem0, ssem1, isem):
        c = lax.axis_index("c")
        s = lax.axis_index("s")
        wid = s * 2 + c
        r0 = s * SUB_ROWS
        sems = (sem0, sem1)
        ssems = (ssem0, ssem1)
        c0 = wid * tile_chunks
        t_iters = jnp.minimum(tile_chunks, jnp.maximum(n_chunks - c0, 0))

        def start_load(b, j):
            pltpu.async_copy(h_hbm.at[pl.ds((c0 + j) * CHUNK, CHUNK)],
                             hbuf.at[b], sems[b])

        def wait_load(b):
            pltpu.make_async_copy(h_hbm.at[pl.ds(0, CHUNK)], hbuf.at[b],
                                  sems[b]).wait()

        pltpu.async_copy(idx_hbm.at[pl.ds(c0, tile_chunks)], idxbuf, isem)
        start_load(0, 0)
        pltpu.sync_copy(z_hbm.at[pl.ds(r0, SUB_ROWS)], acc.at[pl.ds(r0, SUB_ROWS)])
        pltpu.make_async_copy(idx_hbm.at[pl.ds(0, tile_chunks)], idxbuf,
                              isem).wait()
        plsc.subcore_barrier()

        def wait_scat(b, j):
            pltpu.make_async_copy(hbuf.at[b], acc.at[idxbuf.at[j]],
                                  ssems[b]).wait()

        def body(jj, carry):
            for b in (0, 1):
                j = jj * 2 + b

                @pl.when(j < t_iters)
                def _():
                    wait_load(b)

                    @pl.when(j >= 1)
                    def _():
                        wait_scat(b ^ 1, j - 1)

                    @pl.when(j + 1 < t_iters)
                    def _():
                        start_load(b ^ 1, j + 1)

                    pltpu.async_copy(hbuf.at[b], acc.at[idxbuf.at[j]],
                                     ssems[b], add=True)

            return carry

        lax.fori_loop(0, tile_chunks // 2, body, 0)
        last_j = t_iters - 1
        last = last_j % 2

        @pl.when(last == 0)
        def _():
            wait_scat(0, last_j)

        @pl.when(last == 1)
        def _():
            wait_scat(1, last_j)

        plsc.subcore_barrier()
        pltpu.sync_copy(acc.at[pl.ds(r0, SUB_ROWS)],
                        out_hbm.at[pl.ds(c * N_PAD + r0, SUB_ROWS)])

    return run(h, idx2d, zeros)


def kernel(x, rbf, i, num_nodes, W_rbf, W_up, W1, b1, W2, b2, W3, b3, W_out):
    e = x.shape[0]
    nr = rbf.shape[1]
    e_phase = e // P_PHASES
    phase_chunks = e_phase // CHUNK
    tile_chunks = -(-phase_chunks // N_TILES)
    idx_rows = tile_chunks * N_TILES
    bpp = e_phase // EDGE_BLOCK

    rbf_t = jnp.pad(rbf.T, ((0, 8 - nr), (0, 0)))
    wrbf_t = jnp.pad(W_rbf.T, ((0, 8 - nr), (0, 0)))

    idx = i.astype(jnp.int32).reshape(P_PHASES, phase_chunks, CHUNK)
    pad = jnp.full((P_PHASES, idx_rows - phase_chunks, CHUNK), N_PAD - 1,
                   jnp.int32)
    idx3d = jnp.concatenate([idx, pad], axis=1)

    zeros = jnp.zeros((N_PAD, FEAT), jnp.float32)

    partials = []
    for p in range(P_PHASES):
        h_p = pl.pallas_call(
            _edge_body,
            grid=(bpp,),
            in_specs=[
                pl.BlockSpec((8, EDGE_BLOCK), lambda g, p=p: (0, g + p * bpp)),
                pl.BlockSpec((EDGE_BLOCK, FEAT), lambda g, p=p: (g + p * bpp, 0)),
                pl.BlockSpec((8, FEAT), lambda g: (0, 0)),
            ],
            out_specs=pl.BlockSpec((EDGE_BLOCK, FEAT), lambda g: (g, 0)),
            out_shape=jax.ShapeDtypeStruct((e_phase, FEAT), jnp.float32),
        )(rbf_t, x, wrbf_t)
        part = _sc_segment_sum(h_p, idx3d[p], zeros, tile_chunks)
        partials.append(part.reshape(2, N_PAD, FEAT))

    mlp_in_specs = [pl.BlockSpec((2, NODE_BLOCK, FEAT), lambda g: (0, g, 0))
                    for _ in range(P_PHASES)]
    mlp_in_specs += [
        pl.BlockSpec((FEAT, OE), lambda g: (0, 0)),
        pl.BlockSpec((OE, OE), lambda g: (0, 0)),
        pl.BlockSpec((1, OE), lambda g: (0, 0)),
        pl.BlockSpec((OE, OE), lambda g: (0, 0)),
        pl.BlockSpec((1, OE), lambda g: (0, 0)),
        pl.BlockSpec((OE, OE), lambda g: (0, 0)),
        pl.BlockSpec((1, OE), lambda g: (0, 0)),
        pl.BlockSpec((OE, 1), lambda g: (0, 0)),
    ]
    out = pl.pallas_call(
        _mlp_body,
        grid=(N_NODES // NODE_BLOCK,),
        in_specs=mlp_in_specs,
        out_specs=pl.BlockSpec((NODE_BLOCK, 1), lambda g: (g, 0)),
        out_shape=jax.ShapeDtypeStruct((N_NODES, 1), jnp.float32),
    )(*partials, W_up.T, W1.T, b1.reshape(1, OE), W2.T, b2.reshape(1, OE),
      W3.T, b3.reshape(1, OE), W_out.T)
    return out


# speedup vs baseline: 1.0412x; 1.0412x over previous
import functools

import jax
import jax.numpy as jnp
from jax import lax
from jax.experimental import pallas as pl
from jax.experimental.pallas import tpu as pltpu
from jax.experimental.pallas import tpu_sc as plsc

N_NODES = 10000
FEAT = 128
OE = 256
EDGE_BLOCK = 12800
NODE_BLOCK = 2000
CHUNK = 128
N_TILES = 32
N_PAD = 10240
SUB_ROWS = N_PAD // 16
P_PHASES = 1


def _edge_body(rbft_ref, x_ref, w_ref, h_ref):
    s = lax.dot_general(rbft_ref[...], w_ref[...],
                        dimension_numbers=(((0,), (0,)), ((), ())),
                        preferred_element_type=jnp.float32)
    h_ref[...] = s * x_ref[...]


def _mlp_body(*refs):
    p_refs = refs[:P_PHASES]
    wup_ref, w1_ref, b1_ref, w2_ref, b2_ref, w3_ref, b3_ref, wout_ref = \
        refs[P_PHASES:-1]
    o_ref = refs[-1]
    h = p_refs[0][0] + p_refs[0][1]
    for p_ref in p_refs[1:]:
        h = h + p_ref[0] + p_ref[1]
    h = jnp.dot(h, wup_ref[...], preferred_element_type=jnp.float32)
    for w_r, b_r in ((w1_ref, b1_ref), (w2_ref, b2_ref), (w3_ref, b3_ref)):
        z = jnp.dot(h, w_r[...], preferred_element_type=jnp.float32) + b_r[...]
        h = z * jax.nn.sigmoid(z)
    o_ref[...] = jnp.dot(h, wout_ref[...], preferred_element_type=jnp.float32)


def _sc_segment_sum(h, idx2d, zeros, tile_chunks):
    e = h.shape[0]
    n_chunks = e // CHUNK
    mesh = plsc.VectorSubcoreMesh(core_axis_name="c", subcore_axis_name="s")

    @functools.partial(
        pl.kernel,
        mesh=mesh,
        compiler_params=pltpu.CompilerParams(use_tc_tiling_on_sc=True),
        out_type=jax.ShapeDtypeStruct((2 * N_PAD, FEAT), jnp.float32),
        scratch_types=[
            pltpu.VMEM((2, CHUNK, FEAT), jnp.float32),
            pltpu.VMEM((tile_chunks, 128), jnp.int32),
            pltpu.VMEM_SHARED((N_PAD, FEAT), jnp.float32),
            pltpu.SemaphoreType.DMA,
            pltpu.SemaphoreType.DMA,
            pltpu.SemaphoreType.DMA,
            pltpu.SemaphoreType.DMA,
            pltpu.SemaphoreType.DMA,
        ],
    )
    def run(h_hbm, idx_hbm, z_hbm, out_hbm, hbuf, idxbuf, acc,
            sem0, sem1, ssem0, ssem1, isem):
        c = lax.axis_index("c")
        s = lax.axis_index("s")
        wid = s * 2 + c
        r0 = s * SUB_ROWS
        sems = (sem0, sem1)
        ssems = (ssem0, ssem1)
        c0 = wid * tile_chunks
        t_iters = jnp.minimum(tile_chunks, jnp.maximum(n_chunks - c0, 0))

        def start_load(b, j):
            pltpu.async_copy(h_hbm.at[pl.ds((c0 + j) * CHUNK, CHUNK)],
                             hbuf.at[b], sems[b])

        def wait_load(b):
            pltpu.make_async_copy(h_hbm.at[pl.ds(0, CHUNK)], hbuf.at[b],
                                  sems[b]).wait()

        pltpu.async_copy(idx_hbm.at[pl.ds(c0, tile_chunks)], idxbuf, isem)
        start_load(0, 0)
        pltpu.sync_copy(z_hbm.at[pl.ds(r0, SUB_ROWS)], acc.at[pl.ds(r0, SUB_ROWS)])
        pltpu.make_async_copy(idx_hbm.at[pl.ds(0, tile_chunks)], idxbuf,
                              isem).wait()
        plsc.subcore_barrier()

        def wait_scat(b, j):
            pltpu.make_async_copy(hbuf.at[b], acc.at[idxbuf.at[j]],
                                  ssems[b]).wait()

        def body(jj, carry):
            for b in (0, 1):
                j = jj * 2 + b

                @pl.when(j < t_iters)
                def _():
                    wait_load(b)

                    @pl.when(j >= 1)
                    def _():
                        wait_scat(b ^ 1, j - 1)

                    @pl.when(j + 1 < t_iters)
                    def _():
                        start_load(b ^ 1, j + 1)

                    pltpu.async_copy(hbuf.at[b], acc.at[idxbuf.at[j]],
                                     ssems[b], add=True)

            return carry

        lax.fori_loop(0, tile_chunks // 2, body, 0)
        last_j = t_iters - 1
        last = last_j % 2

        @pl.when(last == 0)
        def _():
            wait_scat(0, last_j)

        @pl.when(last == 1)
        def _():
            wait_scat(1, last_j)

        plsc.subcore_barrier()
        pltpu.sync_copy(acc.at[pl.ds(r0, SUB_ROWS)],
                        out_hbm.at[pl.ds(c * N_PAD + r0, SUB_ROWS)])

    return run(h, idx2d, zeros)


def kernel(x, rbf, i, num_nodes, W_rbf, W_up, W1, b1, W2, b2, W3, b3, W_out):
    e = x.shape[0]
    nr = rbf.shape[1]
    e_phase = e // P_PHASES
    phase_chunks = e_phase // CHUNK
    tile_chunks = -(-phase_chunks // N_TILES)
    tile_chunks = -(-tile_chunks // 8) * 8
    idx_rows = tile_chunks * N_TILES
    bpp = e_phase // EDGE_BLOCK

    rbf_t = jnp.pad(rbf.T, ((0, 8 - nr), (0, 0)))
    wrbf_t = jnp.pad(W_rbf.T, ((0, 8 - nr), (0, 0)))

    idx = i.astype(jnp.int32).reshape(P_PHASES, phase_chunks, CHUNK)
    pad = jnp.full((P_PHASES, idx_rows - phase_chunks, CHUNK), N_PAD - 1,
                   jnp.int32)
    idx3d = jnp.concatenate([idx, pad], axis=1)

    zeros = jnp.zeros((N_PAD, FEAT), jnp.float32)

    partials = []
    for p in range(P_PHASES):
        h_p = pl.pallas_call(
            _edge_body,
            grid=(bpp,),
            in_specs=[
                pl.BlockSpec((8, EDGE_BLOCK), lambda g, p=p: (0, g + p * bpp)),
                pl.BlockSpec((EDGE_BLOCK, FEAT), lambda g, p=p: (g + p * bpp, 0)),
                pl.BlockSpec((8, FEAT), lambda g: (0, 0)),
            ],
            out_specs=pl.BlockSpec((EDGE_BLOCK, FEAT), lambda g: (g, 0)),
            out_shape=jax.ShapeDtypeStruct((e_phase, FEAT), jnp.float32),
        )(rbf_t, x, wrbf_t)
        part = _sc_segment_sum(h_p, idx3d[p], zeros, tile_chunks)
        partials.append(part.reshape(2, N_PAD, FEAT))

    mlp_in_specs = [pl.BlockSpec((2, NODE_BLOCK, FEAT), lambda g: (0, g, 0))
                    for _ in range(P_PHASES)]
    mlp_in_specs += [
        pl.BlockSpec((FEAT, OE), lambda g: (0, 0)),
        pl.BlockSpec((OE, OE), lambda g: (0, 0)),
        pl.BlockSpec((1, OE), lambda g: (0, 0)),
        pl.BlockSpec((OE, OE), lambda g: (0, 0)),
        pl.BlockSpec((1, OE), lambda g: (0, 0)),
        pl.BlockSpec((OE, OE), lambda g: (0, 0)),
        pl.BlockSpec((1, OE), lambda g: (0, 0)),
        pl.BlockSpec((OE, 1), lambda g: (0, 0)),
    ]
    out = pl.pallas_call(
        _mlp_body,
        grid=(N_NODES // NODE_BLOCK,),
        in_specs=mlp_in_specs,
        out_specs=pl.BlockSpec((NODE_BLOCK, 1), lambda g: (g, 0)),
        out_shape=jax.ShapeDtypeStruct((N_NODES, 1), jnp.float32),
    )(*partials, W_up.T, W1.T, b1.reshape(1, OE), W2.T, b2.reshape(1, OE),
      W3.T, b3.reshape(1, OE), W_out.T)
    return out
